# baseline (speedup 1.0000x reference)
import functools

import jax
import jax.numpy as jnp
from jax import lax
from jax.experimental import pallas as pl
from jax.experimental.pallas import tpu as pltpu
from jax.experimental.pallas import tpu_sc as plsc

D = 128
S = 50
NB = 4096
NC, NS = 2, 16
NW = NC * NS
CW = NB // NW
HC = 64
NSTEP = 2 * S
M = 12
PF = M // 2
NSG = NSTEP // M
REM = NSTEP - NSG * M

_mesh = plsc.VectorSubcoreMesh(core_axis_name="c", subcore_axis_name="s")


@functools.partial(
    pl.kernel,
    mesh=_mesh,
    out_type=jax.ShapeDtypeStruct((S, NB, D), jnp.float32),
    scratch_types=[
        pltpu.VMEM((S, CW), jnp.int32),
        pltpu.VMEM((M, HC, D), jnp.float32),
    ]
    + [pltpu.SemaphoreType.DMA] * (2 * M),
    compiler_params=pltpu.CompilerParams(use_tc_tiling_on_sc=True),
)
def _embed_gather(xt_hbm, table_hbm, out_hbm, x_v, bufs, *sems):
    in_sems, out_sems = sems[:M], sems[M:]
    wid = lax.axis_index("s") * NC + lax.axis_index("c")
    b0 = wid * CW
    pltpu.sync_copy(xt_hbm.at[:, pl.ds(b0, CW)], x_v)

    def _src(j):
        return table_hbm.at[x_v.at[j // 2, pl.ds((j % 2) * HC, HC)]]

    def _dst(j):
        return out_hbm.at[j // 2, pl.ds(b0 + (j % 2) * HC, HC)]

    def gather(j, b):
        pltpu.async_copy(_src(j), bufs.at[b], in_sems[b])

    def wait_gather(j, b):
        pltpu.make_async_copy(_src(j), bufs.at[b], in_sems[b]).wait()

    def put(j, b):
        pltpu.async_copy(bufs.at[b], _dst(j), out_sems[b])

    def wait_put(j, b):
        pltpu.make_async_copy(bufs.at[b], _dst(j), out_sems[b]).wait()

    for b in range(PF):
        gather(b, b)

    def cycle(sg, carry):
        for b in range(M):
            j = sg * M + b
            wait_gather(j, b)
            put(j, b)
            bp = (b + PF) % M
            jn = j + PF
            if b < PF:

                @pl.when(sg > 0)
                def _():
                    wait_put(jn - M, bp)

                gather(jn, bp)
            else:
                wait_put(j - PF, bp)

                @pl.when(jn < NSTEP)
                def _():
                    gather(jn, bp)

        return carry

    lax.fori_loop(0, NSG, cycle, 0)

    tail = NSG * M
    for r in range(REM):
        j = tail + r
        wait_gather(j, j % M)
        put(j, j % M)
    for j in range(tail - PF, NSTEP):
        wait_put(j, j % M)


def kernel(x, tok_embed):
    out_t = _embed_gather(x.T, tok_embed)
    return out_t.transpose(1, 0, 2)


# speedup vs baseline: 1.0054x; 1.0054x over previous
import functools

import jax
import jax.numpy as jnp
from jax import lax
from jax.experimental import pallas as pl
from jax.experimental.pallas import tpu as pltpu
from jax.experimental.pallas import tpu_sc as plsc

D = 128
S = 50
NB = 4096
NC, NS = 2, 16
NW = NC * NS
CW = NB // NW
HC = 64
NSTEP = 2 * S
M = 12
PF = 8
NSG = NSTEP // M
REM = NSTEP - NSG * M

_mesh = plsc.VectorSubcoreMesh(core_axis_name="c", subcore_axis_name="s")


@functools.partial(
    pl.kernel,
    mesh=_mesh,
    out_type=jax.ShapeDtypeStruct((S, NB, D), jnp.float32),
    scratch_types=[
        pltpu.VMEM((S, CW), jnp.int32),
        pltpu.VMEM((M, HC, D), jnp.float32),
    ]
    + [pltpu.SemaphoreType.DMA] * (2 * M),
    compiler_params=pltpu.CompilerParams(use_tc_tiling_on_sc=True),
)
def _embed_gather(xt_hbm, table_hbm, out_hbm, x_v, bufs, *sems):
    in_sems, out_sems = sems[:M], sems[M:]
    wid = lax.axis_index("s") * NC + lax.axis_index("c")
    b0 = wid * CW
    pltpu.sync_copy(xt_hbm.at[:, pl.ds(b0, CW)], x_v)

    def _src(j):
        return table_hbm.at[x_v.at[j // 2, pl.ds((j % 2) * HC, HC)]]

    def _dst(j):
        return out_hbm.at[j // 2, pl.ds(b0 + (j % 2) * HC, HC)]

    def gather(j, b):
        pltpu.async_copy(_src(j), bufs.at[b], in_sems[b])

    def wait_gather(j, b):
        pltpu.make_async_copy(_src(j), bufs.at[b], in_sems[b]).wait()

    def put(j, b):
        pltpu.async_copy(bufs.at[b], _dst(j), out_sems[b])

    def wait_put(j, b):
        pltpu.make_async_copy(bufs.at[b], _dst(j), out_sems[b]).wait()

    for b in range(PF):
        gather(b, b)

    def cycle(sg, carry):
        for b in range(M):
            j = sg * M + b
            wait_gather(j, b)
            put(j, b)
            bp = (b + PF) % M
            jn = j + PF
            if b < M - PF:

                @pl.when(sg > 0)
                def _():
                    wait_put(jn - M, bp)

                gather(jn, bp)
            else:
                wait_put(jn - M, bp)

                @pl.when(jn < NSTEP)
                def _():
                    gather(jn, bp)

        return carry

    lax.fori_loop(0, NSG, cycle, 0)

    tail = NSG * M
    for r in range(REM):
        j = tail + r
        wait_gather(j, j % M)
        put(j, j % M)
    lastw = NSTEP - REM - 1 - (M - PF)
    for j in range(lastw + 1, NSTEP):
        wait_put(j, j % M)


def kernel(x, tok_embed):
    out_t = _embed_gather(x.T, tok_embed)
    return out_t.transpose(1, 0, 2)
